# baseline (speedup 1.0000x reference)
import functools

import jax
import jax.numpy as jnp
from jax import lax
from jax.experimental import pallas as pl
from jax.experimental.pallas import tpu as pltpu
from jax.experimental.pallas import tpu_sc as plsc

N_NODES = 10000
D = 128
N_CLASSES = 40
E = 320000

NC = 2
NS = 16
NW = NC * NS

N_PAD = 10112
ROWS_PER_TILE = N_PAD // NS

CHUNK = 128
K = 158
E_AGG = NS * K * CHUNK

DCHUNK = 128
DEG_CHUNKS = 158
E_DEG = NS * DEG_CHUNKS * DCHUNK

ZFULL = ROWS_PER_TILE // CHUNK
ZREM = ROWS_PER_TILE % CHUNK
DZFULL = ROWS_PER_TILE // DCHUNK
DZREM = ROWS_PER_TILE % DCHUNK

_mesh2 = plsc.VectorSubcoreMesh(core_axis_name="c", subcore_axis_name="s")
_mesh1 = plsc.VectorSubcoreMesh(core_axis_name="c", subcore_axis_name="s",
                                num_cores=1)


@functools.partial(
    pl.kernel,
    mesh=_mesh2,
    out_type=jax.ShapeDtypeStruct((NC, N_PAD, D), jnp.float32),
    scratch_types=[
        pltpu.VMEM((DEG_CHUNKS, DCHUNK), jnp.int32),
        pltpu.VMEM((DCHUNK, D), jnp.float32),
        pltpu.VMEM_SHARED((N_PAD, D), jnp.float32),
        pltpu.SemaphoreType.DMA,
    ],
)
def _sc_degrees(sd_hbm, out_hbm, idx_v, buf_v, deg_sh, sem):
    c = lax.axis_index("c")
    s = lax.axis_index("s")
    pltpu.sync_copy(sd_hbm.at[c * NS + s], idx_v)

    def zr(i, _):
        buf_v[i // 8, pl.ds((i % 8) * 16, 16)] = jnp.zeros((16,), jnp.float32)
        return 0
    lax.fori_loop(0, DCHUNK * D // 16, zr, 0)
    for k in range(DZFULL):
        pltpu.sync_copy(buf_v,
                        deg_sh.at[pl.ds(s * ROWS_PER_TILE + k * DCHUNK, DCHUNK)])
    pltpu.sync_copy(buf_v.at[pl.ds(0, DZREM)],
                    deg_sh.at[pl.ds(s * ROWS_PER_TILE + DZFULL * DCHUNK, DZREM)])

    def fill(i, _):
        buf_v[i // 8, pl.ds((i % 8) * 16, 16)] = jnp.full((16,), 1.0, jnp.float32)
        return 0
    lax.fori_loop(0, DCHUNK * D // 16, fill, 0)
    plsc.subcore_barrier()

    def body(j, _):
        pltpu.sync_copy(buf_v, deg_sh.at[idx_v.at[j]], add=True)
        return 0
    lax.fori_loop(0, DEG_CHUNKS, body, 0)
    plsc.subcore_barrier()

    pltpu.sync_copy(deg_sh.at[pl.ds(s * ROWS_PER_TILE, ROWS_PER_TILE)],
                    out_hbm.at[c].at[pl.ds(s * ROWS_PER_TILE, ROWS_PER_TILE)])


@functools.partial(
    pl.kernel,
    mesh=_mesh1,
    out_type=jax.ShapeDtypeStruct((N_PAD, D), jnp.float32),
    scratch_types=[
        pltpu.VMEM((2, CHUNK), jnp.int32),
        pltpu.VMEM((2, CHUNK), jnp.int32),
        pltpu.VMEM((2, CHUNK), jnp.int32),
        pltpu.VMEM((2, CHUNK, D), jnp.float32),
        pltpu.VMEM_SHARED((N_PAD, D), jnp.float32),
        pltpu.SemaphoreType.DMA((2,)),
        pltpu.SemaphoreType.DMA((2,)),
    ],
)
def _sc_edge_agg(table_hbm, sd_hbm, out_hbm, sd_ring, src_v, dst_v,
                 rows2, agg_sh, sems, isems):
    s = lax.axis_index("s")

    def unpack(slot):
        def g(k, _):
            p = sd_ring[slot, pl.ds(k * 16, 16)]
            src_v[slot, pl.ds(k * 16, 16)] = jnp.bitwise_and(p, 0xFFFF)
            dst_v[slot, pl.ds(k * 16, 16)] = lax.shift_right_logical(p, 16)
            return 0
        lax.fori_loop(0, CHUNK // 16, g, 0)

    zbuf = rows2.at[0]

    def zr(i, _):
        rows2[0, i // 8, pl.ds((i % 8) * 16, 16)] = jnp.zeros((16,), jnp.float32)
        return 0
    lax.fori_loop(0, CHUNK * D // 16, zr, 0)
    for k in range(ZFULL):
        pltpu.sync_copy(zbuf,
                        agg_sh.at[pl.ds(s * ROWS_PER_TILE + k * CHUNK, CHUNK)])
    pltpu.sync_copy(zbuf.at[pl.ds(0, ZREM)],
                    agg_sh.at[pl.ds(s * ROWS_PER_TILE + ZFULL * CHUNK, ZREM)])
    plsc.subcore_barrier()

    def prologue(b, _):
        pltpu.sync_copy(sd_hbm.at[s].at[b], sd_ring.at[b])
        unpack(b)
        pltpu.async_copy(table_hbm.at[src_v.at[b]], rows2.at[b], sems.at[b])
        return 0
    lax.fori_loop(0, 2, prologue, 0)

    def body(j, _):
        b = lax.rem(j, 2)
        rows = rows2.at[b]
        sem = sems.at[b]

        @pl.when(j + 2 < K)
        def _():
            pltpu.async_copy(sd_hbm.at[s].at[j + 2], sd_ring.at[b], isems.at[b])

        pltpu.make_async_copy(table_hbm.at[src_v.at[b]], rows, sem).wait()
        pltpu.async_copy(rows, agg_sh.at[dst_v.at[b]], sem, add=True)
        pltpu.make_async_copy(rows, agg_sh.at[dst_v.at[b]], sem).wait()

        @pl.when(j + 2 < K)
        def _():
            pltpu.make_async_copy(sd_hbm.at[s].at[j + 2], sd_ring.at[b],
                                  isems.at[b]).wait()
            unpack(b)
            pltpu.async_copy(table_hbm.at[src_v.at[b]], rows, sem)
        return 0
    lax.fori_loop(0, K, body, 0)
    plsc.subcore_barrier()

    pltpu.sync_copy(agg_sh.at[pl.ds(s * ROWS_PER_TILE, ROWS_PER_TILE)],
                    out_hbm.at[pl.ds(s * ROWS_PER_TILE, ROWS_PER_TILE)])


def _tc_prologue_body(x_ref, degs_ref, w_ref, t_ref, onorm_ref, inorm_ref):
    od = degs_ref[0, :, 0:1]
    idg = degs_ref[1, :, 0:1]
    onorm = jnp.where(od > 0, lax.rsqrt(od), 0.0)
    inorm = jnp.where(idg > 0, lax.rsqrt(idg), 0.0)
    onorm_ref[...] = onorm
    inorm_ref[...] = inorm
    t_ref[...] = jnp.dot(x_ref[...] * onorm, w_ref[...],
                         preferred_element_type=jnp.float32)


def _layernorm_relu(p_ref, inorm_ref, b_ref):
    h = p_ref[...] * inorm_ref[...] + b_ref[...]
    h = jnp.maximum(h, 0.0)
    rows = lax.broadcasted_iota(jnp.int32, (N_PAD, D), 0)
    mask = rows < N_NODES
    cnt = float(N_NODES * D)
    mu = jnp.sum(jnp.where(mask, h, 0.0)) / cnt
    var = jnp.sum(jnp.where(mask, (h - mu) ** 2, 0.0)) / cnt
    return (h - mu) * lax.rsqrt(var + 1e-5)


def _tc_mid_body(p_ref, inorm_ref, onorm_ref, b_ref, w_ref, t_ref):
    h = _layernorm_relu(p_ref, inorm_ref, b_ref)
    t_ref[...] = jnp.dot(h * onorm_ref[...], w_ref[...],
                         preferred_element_type=jnp.float32)


def _tc_mid_nomm_body(p_ref, inorm_ref, onorm_ref, b_ref, t_ref):
    h = _layernorm_relu(p_ref, inorm_ref, b_ref)
    t_ref[...] = h * onorm_ref[...]


def _tc_epilogue_body(p_ref, inorm_ref, w_ref, b_ref, out_ref):
    agg = (p_ref[...] * inorm_ref[...])[:N_NODES, :]
    out_ref[...] = jnp.dot(agg, w_ref[...],
                           preferred_element_type=jnp.float32) + b_ref[...]


_tc_prologue = pl.pallas_call(
    _tc_prologue_body,
    out_shape=(jax.ShapeDtypeStruct((N_PAD, D), jnp.float32),
               jax.ShapeDtypeStruct((N_PAD, 1), jnp.float32),
               jax.ShapeDtypeStruct((N_PAD, 1), jnp.float32)),
)

_tc_mid = pl.pallas_call(
    _tc_mid_body,
    out_shape=jax.ShapeDtypeStruct((N_PAD, D), jnp.float32),
)

_tc_mid_nomm = pl.pallas_call(
    _tc_mid_nomm_body,
    out_shape=jax.ShapeDtypeStruct((N_PAD, D), jnp.float32),
)

_tc_epilogue = pl.pallas_call(
    _tc_epilogue_body,
    out_shape=jax.ShapeDtypeStruct((N_NODES, N_CLASSES), jnp.float32),
)


def kernel(x, edge_index, W0, b0, W1, b1, W2, b2):
    src = edge_index[0].astype(jnp.int32)
    dst = edge_index[1].astype(jnp.int32)
    x_pad = jnp.concatenate(
        [x, jnp.zeros((N_PAD - N_NODES, D), jnp.float32)], axis=0)

    pad_a = jnp.full((E_AGG - E,), N_NODES, jnp.int32)
    sd_packed = (jnp.concatenate([src, pad_a])
                 | (jnp.concatenate([dst, pad_a]) << 16)
                 ).reshape(NS, K, CHUNK)

    pad_d = jnp.full((E_DEG - E,), N_NODES, jnp.int32)
    sd = jnp.concatenate([src, pad_d, dst, pad_d]).reshape(
        NW, DEG_CHUNKS, DCHUNK)

    degs = _sc_degrees(sd)
    t0, onorm, inorm = _tc_prologue(x_pad, degs, W0)
    p0 = _sc_edge_agg(t0, sd_packed)
    t1 = _tc_mid(p0, inorm, onorm, b0.reshape(1, D), W1)
    p1 = _sc_edge_agg(t1, sd_packed)
    t2 = _tc_mid_nomm(p1, inorm, onorm, b1.reshape(1, D))
    p2 = _sc_edge_agg(t2, sd_packed)
    return _tc_epilogue(p2, inorm, W2, b2.reshape(1, N_CLASSES))


# speedup vs baseline: 1.2155x; 1.2155x over previous
import functools

import jax
import jax.numpy as jnp
from jax import lax
from jax.experimental import pallas as pl
from jax.experimental.pallas import tpu as pltpu
from jax.experimental.pallas import tpu_sc as plsc

N_NODES = 10000
D = 128
N_CLASSES = 40
E = 320000

NC = 2
NS = 16
NW = NC * NS

N_PAD = 10112
ROWS_PER_TILE = N_PAD // NS

CHUNK = 128
K0 = 120
K1 = 38
KMAX = max(K0, K1)
E_AGG = NS * (K0 + K1) * CHUNK

DCHUNK = 128
DEG_CHUNKS = 158
E_DEG = NS * DEG_CHUNKS * DCHUNK

ZFULL = ROWS_PER_TILE // CHUNK
ZREM = ROWS_PER_TILE % CHUNK
DZFULL = ROWS_PER_TILE // DCHUNK
DZREM = ROWS_PER_TILE % DCHUNK

_mesh2 = plsc.VectorSubcoreMesh(core_axis_name="c", subcore_axis_name="s")
_mesh1 = plsc.VectorSubcoreMesh(core_axis_name="c", subcore_axis_name="s",
                                num_cores=1)


@functools.partial(
    pl.kernel,
    mesh=_mesh2,
    out_type=jax.ShapeDtypeStruct((NC, N_PAD, D), jnp.float32),
    scratch_types=[
        pltpu.VMEM((DEG_CHUNKS, DCHUNK), jnp.int32),
        pltpu.VMEM((DCHUNK, D), jnp.float32),
        pltpu.VMEM_SHARED((N_PAD, D), jnp.float32),
        pltpu.SemaphoreType.DMA,
    ],
)
def _sc_degrees(sd_hbm, out_hbm, idx_v, buf_v, deg_sh, sem):
    c = lax.axis_index("c")
    s = lax.axis_index("s")
    pltpu.sync_copy(sd_hbm.at[c * NS + s], idx_v)

    def zr(i, _):
        buf_v[i // 8, pl.ds((i % 8) * 16, 16)] = jnp.zeros((16,), jnp.float32)
        return 0
    lax.fori_loop(0, DCHUNK * D // 16, zr, 0)
    for k in range(DZFULL):
        pltpu.sync_copy(buf_v,
                        deg_sh.at[pl.ds(s * ROWS_PER_TILE + k * DCHUNK, DCHUNK)])
    pltpu.sync_copy(buf_v.at[pl.ds(0, DZREM)],
                    deg_sh.at[pl.ds(s * ROWS_PER_TILE + DZFULL * DCHUNK, DZREM)])

    def fill(i, _):
        buf_v[i // 8, pl.ds((i % 8) * 16, 16)] = jnp.full((16,), 1.0, jnp.float32)
        return 0
    lax.fori_loop(0, DCHUNK * D // 16, fill, 0)
    plsc.subcore_barrier()

    def body(j, _):
        pltpu.sync_copy(buf_v, deg_sh.at[idx_v.at[j]], add=True)
        return 0
    lax.fori_loop(0, DEG_CHUNKS, body, 0)
    plsc.subcore_barrier()

    pltpu.sync_copy(deg_sh.at[pl.ds(s * ROWS_PER_TILE, ROWS_PER_TILE)],
                    out_hbm.at[c].at[pl.ds(s * ROWS_PER_TILE, ROWS_PER_TILE)])


@functools.partial(
    pl.kernel,
    mesh=_mesh2,
    out_type=jax.ShapeDtypeStruct((NC, N_PAD, D), jnp.float32),
    scratch_types=[
        pltpu.VMEM((KMAX, CHUNK), jnp.int32),
        pltpu.VMEM((2, CHUNK), jnp.int32),
        pltpu.VMEM((2, CHUNK), jnp.int32),
        pltpu.VMEM((2, CHUNK, D), jnp.float32),
        pltpu.VMEM_SHARED((N_PAD, D), jnp.float32),
        pltpu.SemaphoreType.DMA((2,)),
    ],
)
def _sc_edge_agg(table_hbm, sd_hbm, out_hbm, sd_v, src_v, dst_v,
                 rows2, agg_sh, sems):
    c = lax.axis_index("c")
    s = lax.axis_index("s")
    wid = c * NS + s
    nch = jnp.where(c == 0, K0, K1)
    pltpu.sync_copy(sd_hbm.at[wid], sd_v)

    def unpack(ch, slot):
        def g(k, _):
            p = sd_v[ch, pl.ds(k * 16, 16)]
            src_v[slot, pl.ds(k * 16, 16)] = jnp.bitwise_and(p, 0xFFFF)
            dst_v[slot, pl.ds(k * 16, 16)] = lax.shift_right_logical(p, 16)
            return 0
        lax.fori_loop(0, CHUNK // 16, g, 0)

    zbuf = rows2.at[0]

    def zr(i, _):
        rows2[0, i // 8, pl.ds((i % 8) * 16, 16)] = jnp.zeros((16,), jnp.float32)
        return 0
    lax.fori_loop(0, CHUNK * D // 16, zr, 0)
    for k in range(ZFULL):
        pltpu.sync_copy(zbuf,
                        agg_sh.at[pl.ds(s * ROWS_PER_TILE + k * CHUNK, CHUNK)])
    pltpu.sync_copy(zbuf.at[pl.ds(0, ZREM)],
                    agg_sh.at[pl.ds(s * ROWS_PER_TILE + ZFULL * CHUNK, ZREM)])
    plsc.subcore_barrier()

    def prologue(b, _):
        unpack(b, b)
        pltpu.async_copy(table_hbm.at[src_v.at[b]], rows2.at[b], sems.at[b])
        return 0
    lax.fori_loop(0, 2, prologue, 0)

    def body(j, _):
        b = lax.rem(j, 2)
        rows = rows2.at[b]
        sem = sems.at[b]
        pltpu.make_async_copy(table_hbm.at[src_v.at[b]], rows, sem).wait()
        pltpu.async_copy(rows, agg_sh.at[dst_v.at[b]], sem, add=True)
        pltpu.make_async_copy(rows, agg_sh.at[dst_v.at[b]], sem).wait()

        @pl.when(j + 2 < nch)
        def _():
            unpack(j + 2, b)
            pltpu.async_copy(table_hbm.at[src_v.at[b]], rows, sem)
        return 0
    lax.fori_loop(0, nch, body, 0)
    plsc.subcore_barrier()

    pltpu.sync_copy(agg_sh.at[pl.ds(s * ROWS_PER_TILE, ROWS_PER_TILE)],
                    out_hbm.at[c].at[pl.ds(s * ROWS_PER_TILE, ROWS_PER_TILE)])


def _tc_prologue_body(x_ref, degs_ref, w_ref, t_ref, onorm_ref, inorm_ref):
    od = degs_ref[0, :, 0:1]
    idg = degs_ref[1, :, 0:1]
    onorm = jnp.where(od > 0, lax.rsqrt(od), 0.0)
    inorm = jnp.where(idg > 0, lax.rsqrt(idg), 0.0)
    onorm_ref[...] = onorm
    inorm_ref[...] = inorm
    t_ref[...] = jnp.dot(x_ref[...] * onorm, w_ref[...],
                         preferred_element_type=jnp.float32)


def _layernorm_relu(p_ref, inorm_ref, b_ref):
    h = (p_ref[0] + p_ref[1]) * inorm_ref[...] + b_ref[...]
    h = jnp.maximum(h, 0.0)
    rows = lax.broadcasted_iota(jnp.int32, (N_PAD, D), 0)
    mask = rows < N_NODES
    cnt = float(N_NODES * D)
    mu = jnp.sum(jnp.where(mask, h, 0.0)) / cnt
    var = jnp.sum(jnp.where(mask, (h - mu) ** 2, 0.0)) / cnt
    return (h - mu) * lax.rsqrt(var + 1e-5)


def _tc_mid_body(p_ref, inorm_ref, onorm_ref, b_ref, w_ref, t_ref):
    h = _layernorm_relu(p_ref, inorm_ref, b_ref)
    t_ref[...] = jnp.dot(h * onorm_ref[...], w_ref[...],
                         preferred_element_type=jnp.float32)


def _tc_mid_nomm_body(p_ref, inorm_ref, onorm_ref, b_ref, t_ref):
    h = _layernorm_relu(p_ref, inorm_ref, b_ref)
    t_ref[...] = h * onorm_ref[...]


def _tc_epilogue_body(p_ref, inorm_ref, w_ref, b_ref, out_ref):
    agg = ((p_ref[0] + p_ref[1]) * inorm_ref[...])[:N_NODES, :]
    out_ref[...] = jnp.dot(agg, w_ref[...],
                           preferred_element_type=jnp.float32) + b_ref[...]


_tc_prologue = pl.pallas_call(
    _tc_prologue_body,
    out_shape=(jax.ShapeDtypeStruct((N_PAD, D), jnp.float32),
               jax.ShapeDtypeStruct((N_PAD, 1), jnp.float32),
               jax.ShapeDtypeStruct((N_PAD, 1), jnp.float32)),
)

_tc_mid = pl.pallas_call(
    _tc_mid_body,
    out_shape=jax.ShapeDtypeStruct((N_PAD, D), jnp.float32),
)

_tc_mid_nomm = pl.pallas_call(
    _tc_mid_nomm_body,
    out_shape=jax.ShapeDtypeStruct((N_PAD, D), jnp.float32),
)

_tc_epilogue = pl.pallas_call(
    _tc_epilogue_body,
    out_shape=jax.ShapeDtypeStruct((N_NODES, N_CLASSES), jnp.float32),
)


def kernel(x, edge_index, W0, b0, W1, b1, W2, b2):
    src = edge_index[0].astype(jnp.int32)
    dst = edge_index[1].astype(jnp.int32)
    x_pad = jnp.concatenate(
        [x, jnp.zeros((N_PAD - N_NODES, D), jnp.float32)], axis=0)

    pad_a = jnp.full((E_AGG - E,), N_NODES, jnp.int32)
    flat = (jnp.concatenate([src, pad_a])
            | (jnp.concatenate([dst, pad_a]) << 16))
    n0 = NS * K0 * CHUNK
    sd0 = flat[:n0].reshape(NS, K0, CHUNK)
    sd1 = flat[n0:].reshape(NS, K1, CHUNK)
    fillv = jnp.int32(N_NODES | (N_NODES << 16))
    if K0 < KMAX:
        sd0 = jnp.concatenate(
            [sd0, jnp.full((NS, KMAX - K0, CHUNK), fillv)], axis=1)
    if K1 < KMAX:
        sd1 = jnp.concatenate(
            [sd1, jnp.full((NS, KMAX - K1, CHUNK), fillv)], axis=1)
    sd_packed = jnp.concatenate([sd0, sd1], axis=0)

    pad_d = jnp.full((E_DEG - E,), N_NODES, jnp.int32)
    sd = jnp.concatenate([src, pad_d, dst, pad_d]).reshape(
        NW, DEG_CHUNKS, DCHUNK)

    degs = _sc_degrees(sd)
    t0, onorm, inorm = _tc_prologue(x_pad, degs, W0)
    p0 = _sc_edge_agg(t0, sd_packed)
    t1 = _tc_mid(p0, inorm, onorm, b0.reshape(1, D), W1)
    p1 = _sc_edge_agg(t1, sd_packed)
    t2 = _tc_mid_nomm(p1, inorm, onorm, b1.reshape(1, D))
    p2 = _sc_edge_agg(t2, sd_packed)
    return _tc_epilogue(p2, inorm, W2, b2.reshape(1, N_CLASSES))
